# Initial kernel scaffold; baseline (speedup 1.0000x reference)
import jax
import jax.numpy as jnp
from jax.experimental import pallas as pl


def kernel(x, embeddings, W_in, b_in, W_out, b_out):
    raise NotImplementedError("write your pallas kernel here")


# speedup vs baseline: 38.9799x; 38.9799x over previous
import functools

import jax
import jax.numpy as jnp
from jax import lax
from jax.experimental import pallas as pl
from jax.experimental.pallas import tpu as pltpu
from jax.experimental.pallas import tpu_sc as plsc

VOCAB = 1000
EMB = 128
VEC = 64
B = 16384
L = 200

NC = 2
NS = 16
LN = 16
NW = NC * NS
ROWS_PER_W = B // NW
R = 32
NCHUNK = ROWS_PER_W // R
CW = R * L
CV = R * VOCAB

_mesh = plsc.VectorSubcoreMesh(core_axis_name="c", subcore_axis_name="s")


@functools.partial(
    pl.kernel,
    out_type=jax.ShapeDtypeStruct((B * VOCAB,), jnp.float32),
    mesh=_mesh,
    scratch_types=[
        pltpu.VMEM((CW,), jnp.int32),
        pltpu.VMEM((CV,), jnp.float32),
    ],
    compiler_params=pltpu.CompilerParams(needs_layout_passes=False),
)
def _sc_counts(x_hbm, out_hbm, idx_v, cnt_v):
    wid = lax.axis_index("s") * NC + lax.axis_index("c")
    ones = jnp.ones((LN,), jnp.float32)
    zeros = jnp.zeros((LN,), jnp.float32)
    lane = lax.iota(jnp.int32, LN)

    def zbody(i, carry):
        cnt_v[pl.ds(i * LN, LN)] = zeros
        return carry

    lax.fori_loop(0, CV // LN, zbody, 0)

    def make_pair_pass(value_vec, add):
        def pair_pass(p, carry):
            off0 = (2 * p) * VOCAB
            off1 = off0 + VOCAB
            moff = jnp.where(lane < 8, off0, off1)
            for j in range(25):
                xv = idx_v[pl.ds(p * (2 * L) + j * LN, LN)]
                if j < 12:
                    fidx = xv + off0
                elif j == 12:
                    fidx = xv + moff
                else:
                    fidx = xv + off1
                if add:
                    plsc.addupdate_scatter(cnt_v, [fidx], value_vec)
                else:
                    plsc.store_scatter(cnt_v, [fidx], value_vec)
            return carry

        return pair_pass

    def chunk_body(g, carry):
        row0 = wid * ROWS_PER_W + g * R
        pltpu.sync_copy(x_hbm.at[pl.ds(row0 * L, CW)], idx_v)
        lax.fori_loop(0, R // 2, make_pair_pass(ones, True), 0)
        pltpu.sync_copy(cnt_v, out_hbm.at[pl.ds(row0 * VOCAB, CV)])
        lax.fori_loop(0, R // 2, make_pair_pass(zeros, False), 0)
        return carry

    lax.fori_loop(0, NCHUNK, chunk_body, 0)


def _prep_body(emb_ref, win_ref, bin_ref, wout_ref, bout_ref, t_ref, c_ref):
    t_ref[...] = lax.dot_general(
        emb_ref[...], win_ref[...], (((1,), (1,)), ((), ())),
        preferred_element_type=jnp.float32)
    c_ref[...] = lax.dot_general(
        bin_ref[...], wout_ref[...], (((1,), (1,)), ((), ())),
        preferred_element_type=jnp.float32) + bout_ref[...]


BM = 1024


def _main_body(cnt_ref, t_ref, wout_ref, c_ref, out_ref):
    h = jnp.dot(cnt_ref[...], t_ref[...], preferred_element_type=jnp.float32)
    out_ref[...] = lax.dot_general(
        h, wout_ref[...], (((1,), (1,)), ((), ())),
        preferred_element_type=jnp.float32) + c_ref[...]


def kernel(x, embeddings, W_in, b_in, W_out, b_out):
    counts = _sc_counts(x.reshape(B * L))
    t, c = pl.pallas_call(
        _prep_body,
        out_shape=(
            jax.ShapeDtypeStruct((VOCAB, VEC), jnp.float32),
            jax.ShapeDtypeStruct((1, VOCAB), jnp.float32),
        ),
    )(embeddings, W_in, b_in.reshape(1, VEC), W_out, b_out.reshape(1, VOCAB))
    out = pl.pallas_call(
        _main_body,
        grid=(B // BM,),
        in_specs=[
            pl.BlockSpec((BM, VOCAB), lambda i: (i, 0)),
            pl.BlockSpec((VOCAB, VEC), lambda i: (0, 0)),
            pl.BlockSpec((VOCAB, VEC), lambda i: (0, 0)),
            pl.BlockSpec((1, VOCAB), lambda i: (0, 0)),
        ],
        out_specs=pl.BlockSpec((BM, VOCAB), lambda i: (i, 0)),
        out_shape=jax.ShapeDtypeStruct((B, VOCAB), jnp.float32),
    )(counts.reshape(B, VOCAB), t, W_out, c)
    return out


# speedup vs baseline: 44.6765x; 1.1461x over previous
import functools

import jax
import jax.numpy as jnp
from jax import lax
from jax.experimental import pallas as pl
from jax.experimental.pallas import tpu as pltpu
from jax.experimental.pallas import tpu_sc as plsc

VOCAB = 1000
VPAD = 1024
EMB = 128
VEC = 64
B = 16384
L = 200

NC = 2
NS = 16
LN = 16
NW = NC * NS
ROWS_PER_W = B // NW
R = 32
NCHUNK = ROWS_PER_W // R
CV = R * VPAD

_mesh = plsc.VectorSubcoreMesh(core_axis_name="c", subcore_axis_name="s")


@functools.partial(
    pl.kernel,
    out_type=jax.ShapeDtypeStruct((B * VPAD,), jnp.float32),
    mesh=_mesh,
    scratch_types=[
        pltpu.VMEM((R, L), jnp.int32),
        pltpu.VMEM((R, L), jnp.int32),
        pltpu.VMEM((CV,), jnp.float32),
        pltpu.VMEM((CV,), jnp.float32),
        pltpu.SemaphoreType.DMA,
        pltpu.SemaphoreType.DMA,
    ],
    compiler_params=pltpu.CompilerParams(needs_layout_passes=False),
)
def _sc_counts(x_hbm, out_hbm, idx0, idx1, cnt0, cnt1, sem0, sem1):
    wid = lax.axis_index("s") * NC + lax.axis_index("c")
    ones = jnp.ones((LN,), jnp.float32)
    zeros = jnp.zeros((LN,), jnp.float32)
    lane = lax.iota(jnp.int32, LN)
    tail_mask = lane >= 8
    idx_b = (idx0, idx1)
    cnt_b = (cnt0, cnt1)
    sem_b = (sem0, sem1)

    def zbody(i, carry):
        cnt0[pl.ds(i * LN, LN)] = zeros
        cnt1[pl.ds(i * LN, LN)] = zeros
        return carry

    lax.fori_loop(0, CV // LN, zbody, 0)

    def row_pass(slot, r, value_vec, add):
        roff = r * VPAD
        for j in range(13):
            if j < 12:
                xv = idx_b[slot][r, pl.ds(j * LN, LN)]
                mask = None
            else:
                xv = idx_b[slot][r, pl.ds(L - LN, LN)]
                mask = tail_mask
            fidx = xv + roff
            if add:
                plsc.addupdate_scatter(cnt_b[slot], [fidx], value_vec, mask=mask)
            else:
                plsc.store_scatter(cnt_b[slot], [fidx], zeros, mask=mask)

    def add_pass(slot):
        def body(r, carry):
            row_pass(slot, r, ones, True)
            return carry
        lax.fori_loop(0, R, body, 0)

    def zero_pass(slot):
        def body(r, carry):
            row_pass(slot, r, zeros, False)
            return carry
        lax.fori_loop(0, R, body, 0)

    def out_copy(slot, g):
        row0 = wid * ROWS_PER_W + g * R
        return pltpu.make_async_copy(
            cnt_b[slot], out_hbm.at[pl.ds(row0 * VPAD, CV)], sem_b[slot])

    def process(slot, g):
        row0 = wid * ROWS_PER_W + g * R
        pltpu.sync_copy(x_hbm.at[pl.ds(row0, R)], idx_b[slot])
        add_pass(slot)
        out_copy(slot, g).start()

    process(0, 0)
    process(1, 1)

    def chunk_body(g, carry):
        def for_slot(slot):
            @pl.when(lax.rem(g, 2) == slot)
            def _():
                out_copy(slot, g - 2).wait()
                zero_pass(slot)
                process(slot, g)
        for_slot(0)
        for_slot(1)
        return carry

    lax.fori_loop(2, NCHUNK, chunk_body, 0)
    out_copy(0, NCHUNK - 2).wait()
    out_copy(1, NCHUNK - 1).wait()


def _prep_body(emb_ref, win_ref, bin_ref, wout_ref, bout_ref, t_ref, c_ref):
    t_ref[pl.ds(0, VOCAB), :] = lax.dot_general(
        emb_ref[...], win_ref[...], (((1,), (1,)), ((), ())),
        preferred_element_type=jnp.float32)
    t_ref[pl.ds(VOCAB, VPAD - VOCAB), :] = jnp.zeros(
        (VPAD - VOCAB, VEC), jnp.float32)
    c_ref[...] = lax.dot_general(
        bin_ref[...], wout_ref[...], (((1,), (1,)), ((), ())),
        preferred_element_type=jnp.float32) + bout_ref[...]


BM = 1024


def _main_body(cnt_ref, t_ref, wout_ref, c_ref, out_ref):
    h = jnp.dot(cnt_ref[...], t_ref[...], preferred_element_type=jnp.float32)
    out_ref[...] = lax.dot_general(
        h, wout_ref[...], (((1,), (1,)), ((), ())),
        preferred_element_type=jnp.float32) + c_ref[...]


def kernel(x, embeddings, W_in, b_in, W_out, b_out):
    counts = _sc_counts(x)
    t, c = pl.pallas_call(
        _prep_body,
        out_shape=(
            jax.ShapeDtypeStruct((VPAD, VEC), jnp.float32),
            jax.ShapeDtypeStruct((1, VOCAB), jnp.float32),
        ),
    )(embeddings, W_in, b_in.reshape(1, VEC), W_out, b_out.reshape(1, VOCAB))
    out = pl.pallas_call(
        _main_body,
        grid=(B // BM,),
        in_specs=[
            pl.BlockSpec((BM, VPAD), lambda i: (i, 0)),
            pl.BlockSpec((VPAD, VEC), lambda i: (0, 0)),
            pl.BlockSpec((VOCAB, VEC), lambda i: (0, 0)),
            pl.BlockSpec((1, VOCAB), lambda i: (0, 0)),
        ],
        out_specs=pl.BlockSpec((BM, VOCAB), lambda i: (i, 0)),
        out_shape=jax.ShapeDtypeStruct((B, VOCAB), jnp.float32),
    )(counts.reshape(B, VPAD), t, W_out, c)
    return out


# speedup vs baseline: 54.6061x; 1.2223x over previous
import functools

import jax
import jax.numpy as jnp
from jax import lax
from jax.experimental import pallas as pl
from jax.experimental.pallas import tpu as pltpu
from jax.experimental.pallas import tpu_sc as plsc

VOCAB = 1000
VPAD = 1024
EMB = 128
VEC = 64
B = 16384
L = 200

NC = 2
NS = 16
LN = 16
NW = NC * NS
ROWS_PER_W = B // NW
R = 32
NCHUNK = ROWS_PER_W // R
CV = R * VPAD

_mesh = plsc.VectorSubcoreMesh(core_axis_name="c", subcore_axis_name="s")


@functools.partial(
    pl.kernel,
    out_type=jax.ShapeDtypeStruct((B * VPAD,), jnp.float32),
    mesh=_mesh,
    scratch_types=[
        pltpu.VMEM((R, L), jnp.int32),
        pltpu.VMEM((R, L), jnp.int32),
        pltpu.VMEM((CV,), jnp.float32),
        pltpu.VMEM((CV,), jnp.float32),
        pltpu.SemaphoreType.DMA,
        pltpu.SemaphoreType.DMA,
    ],
    compiler_params=pltpu.CompilerParams(needs_layout_passes=False),
)
def _sc_counts(x_hbm, out_hbm, idx0, idx1, cnt0, cnt1, sem0, sem1):
    wid = lax.axis_index("s") * NC + lax.axis_index("c")
    ones = jnp.ones((LN,), jnp.float32)
    zeros = jnp.zeros((LN,), jnp.float32)
    lane = lax.iota(jnp.int32, LN)
    tail_mask = lane >= 8
    idx_b = (idx0, idx1)
    cnt_b = (cnt0, cnt1)
    sem_b = (sem0, sem1)

    def zbody(i, carry):
        cnt0[pl.ds(i * LN, LN)] = zeros
        cnt1[pl.ds(i * LN, LN)] = zeros
        return carry

    lax.fori_loop(0, CV // LN, zbody, 0)

    def row_pass(slot, r, value_vec, add):
        roff = r * VPAD
        for j in range(13):
            if j < 12:
                xv = idx_b[slot][r, pl.ds(j * LN, LN)]
                mask = None
            else:
                xv = idx_b[slot][r, pl.ds(L - LN, LN)]
                mask = tail_mask
            fidx = xv + roff
            if add:
                plsc.addupdate_scatter(cnt_b[slot], [fidx], value_vec, mask=mask)
            else:
                plsc.store_scatter(cnt_b[slot], [fidx], zeros, mask=mask)

    def add_pass(slot):
        def body(r, carry):
            row_pass(slot, r, ones, True)
            return carry
        lax.fori_loop(0, R, body, 0)

    def zero_pass(slot):
        def body(r, carry):
            row_pass(slot, r, zeros, False)
            return carry
        lax.fori_loop(0, R, body, 0)

    def out_copy(slot, g):
        row0 = wid * ROWS_PER_W + g * R
        return pltpu.make_async_copy(
            cnt_b[slot], out_hbm.at[pl.ds(row0 * VPAD, CV)], sem_b[slot])

    def process(slot, g):
        row0 = wid * ROWS_PER_W + g * R
        pltpu.sync_copy(x_hbm.at[pl.ds(row0, R)], idx_b[slot])
        add_pass(slot)
        out_copy(slot, g).start()

    process(0, 0)
    process(1, 1)

    def chunk_body(g, carry):
        def for_slot(slot):
            @pl.when(lax.rem(g, 2) == slot)
            def _():
                out_copy(slot, g - 2).wait()
                zero_pass(slot)
                process(slot, g)
        for_slot(0)
        for_slot(1)
        return carry

    lax.fori_loop(2, NCHUNK, chunk_body, 0)
    out_copy(0, NCHUNK - 2).wait()
    out_copy(1, NCHUNK - 1).wait()


def _prep_body(emb_ref, win_ref, bin_ref, wout_ref, bout_ref, t_ref, c_ref):
    t_ref[pl.ds(0, VOCAB), :] = lax.dot_general(
        emb_ref[...], win_ref[...], (((1,), (1,)), ((), ())),
        preferred_element_type=jnp.float32)
    t_ref[pl.ds(VOCAB, VPAD - VOCAB), :] = jnp.zeros(
        (VPAD - VOCAB, VEC), jnp.float32)
    c_ref[...] = lax.dot_general(
        bin_ref[...], wout_ref[...], (((1,), (1,)), ((), ())),
        preferred_element_type=jnp.float32) + bout_ref[...]


BM = 1024


def _main_body(cnt_ref, t_ref, wout_ref, c_ref, out_ref):
    h = jnp.dot(cnt_ref[:, 0, :], t_ref[pl.ds(0, 128), :],
                preferred_element_type=jnp.float32)
    for j in range(1, 8):
        h = h + jnp.dot(cnt_ref[:, j, :], t_ref[pl.ds(j * 128, 128), :],
                        preferred_element_type=jnp.float32)
    out_ref[...] = lax.dot_general(
        h, wout_ref[...], (((1,), (1,)), ((), ())),
        preferred_element_type=jnp.float32) + c_ref[...]


def kernel(x, embeddings, W_in, b_in, W_out, b_out):
    counts = _sc_counts(x)
    t, c = pl.pallas_call(
        _prep_body,
        out_shape=(
            jax.ShapeDtypeStruct((VPAD, VEC), jnp.float32),
            jax.ShapeDtypeStruct((1, VOCAB), jnp.float32),
        ),
    )(embeddings, W_in, b_in.reshape(1, VEC), W_out, b_out.reshape(1, VOCAB))
    out = pl.pallas_call(
        _main_body,
        grid=(B // BM,),
        in_specs=[
            pl.BlockSpec((BM, 8, 128), lambda i: (i, 0, 0)),
            pl.BlockSpec((VPAD, VEC), lambda i: (0, 0)),
            pl.BlockSpec((VOCAB, VEC), lambda i: (0, 0)),
            pl.BlockSpec((1, VOCAB), lambda i: (0, 0)),
        ],
        out_specs=pl.BlockSpec((BM, VOCAB), lambda i: (i, 0)),
        out_shape=jax.ShapeDtypeStruct((B, VOCAB), jnp.float32),
    )(counts.reshape(B, 8, 128), t, W_out, c)
    return out


# speedup vs baseline: 70.1424x; 1.2845x over previous
import functools

import jax
import jax.numpy as jnp
from jax import lax
from jax.experimental import pallas as pl
from jax.experimental.pallas import tpu as pltpu
from jax.experimental.pallas import tpu_sc as plsc

VOCAB = 1000
VPAD = 1024
EMB = 128
VEC = 64
B = 16384
L = 200

NC = 2
NS = 16
LN = 16
NW = NC * NS
ROWS_PER_W = B // NW
R = 32
NCHUNK = ROWS_PER_W // R
CV = R * VPAD

_mesh = plsc.VectorSubcoreMesh(core_axis_name="c", subcore_axis_name="s")


@functools.partial(
    pl.kernel,
    out_type=jax.ShapeDtypeStruct((B * VPAD,), jnp.float32),
    mesh=_mesh,
    scratch_types=[
        pltpu.VMEM((R, L), jnp.int32),
        pltpu.VMEM((R, L), jnp.int32),
        pltpu.VMEM((CV,), jnp.float32),
        pltpu.VMEM((CV,), jnp.float32),
        pltpu.SemaphoreType.DMA,
        pltpu.SemaphoreType.DMA,
    ],
    compiler_params=pltpu.CompilerParams(needs_layout_passes=False),
)
def _sc_counts(x_hbm, out_hbm, idx0, idx1, cnt0, cnt1, sem0, sem1):
    wid = lax.axis_index("s") * NC + lax.axis_index("c")
    ones = jnp.ones((LN,), jnp.float32)
    zeros = jnp.zeros((LN,), jnp.float32)
    lane = lax.iota(jnp.int32, LN)
    tail_mask = lane >= 8
    idx_b = (idx0, idx1)
    cnt_b = (cnt0, cnt1)
    sem_b = (sem0, sem1)

    def zbody(i, carry):
        cnt0[pl.ds(i * LN, LN)] = zeros
        cnt1[pl.ds(i * LN, LN)] = zeros
        return carry

    lax.fori_loop(0, CV // LN, zbody, 0)

    bias_idx0 = lane * VPAD + VOCAB
    bias_idx1 = bias_idx0 + LN * VPAD
    for cbuf in (cnt0, cnt1):
        plsc.store_scatter(cbuf, [bias_idx0], ones)
        plsc.store_scatter(cbuf, [bias_idx1], ones)

    def row_pass(slot, r, value_vec, add):
        roff = r * VPAD
        for j in range(13):
            if j < 12:
                xv = idx_b[slot][r, pl.ds(j * LN, LN)]
                mask = None
            else:
                xv = idx_b[slot][r, pl.ds(L - LN, LN)]
                mask = tail_mask
            fidx = xv + roff
            if add:
                plsc.addupdate_scatter(cnt_b[slot], [fidx], value_vec, mask=mask)
            else:
                plsc.store_scatter(cnt_b[slot], [fidx], zeros, mask=mask)

    def add_pass(slot):
        def body(r, carry):
            row_pass(slot, r, ones, True)
            return carry
        lax.fori_loop(0, R, body, 0)

    def zero_pass(slot):
        def body(r, carry):
            row_pass(slot, r, zeros, False)
            return carry
        lax.fori_loop(0, R, body, 0)

    def out_copy(slot, g):
        row0 = wid * ROWS_PER_W + g * R
        return pltpu.make_async_copy(
            cnt_b[slot], out_hbm.at[pl.ds(row0 * VPAD, CV)], sem_b[slot])

    def process(slot, g):
        row0 = wid * ROWS_PER_W + g * R
        pltpu.sync_copy(x_hbm.at[pl.ds(row0, R)], idx_b[slot])
        add_pass(slot)
        out_copy(slot, g).start()

    process(0, 0)
    process(1, 1)

    def chunk_body(g, carry):
        def for_slot(slot):
            @pl.when(lax.rem(g, 2) == slot)
            def _():
                out_copy(slot, g - 2).wait()
                zero_pass(slot)
                process(slot, g)
        for_slot(0)
        for_slot(1)
        return carry

    lax.fori_loop(2, NCHUNK, chunk_body, 0)
    out_copy(0, NCHUNK - 2).wait()
    out_copy(1, NCHUNK - 1).wait()


def _prep_body(emb_ref, win_ref, bin_ref, t_ref):
    t_ref[pl.ds(0, VOCAB), :] = lax.dot_general(
        emb_ref[...], win_ref[...], (((1,), (1,)), ((), ())),
        preferred_element_type=jnp.float32)
    t_ref[pl.ds(VOCAB, VPAD - VOCAB), :] = jnp.zeros(
        (VPAD - VOCAB, VEC), jnp.float32)
    t_ref[pl.ds(VOCAB, 1), :] = bin_ref[...]


BM = 1024


def _main_body(cnt_ref, t_ref, wout_ref, bout_ref, out_ref):
    h = jnp.dot(cnt_ref[:, 0, :], t_ref[pl.ds(0, 128), :],
                preferred_element_type=jnp.float32)
    for j in range(1, 8):
        h = h + jnp.dot(cnt_ref[:, j, :], t_ref[pl.ds(j * 128, 128), :],
                        preferred_element_type=jnp.float32)
    e0 = (lax.broadcasted_iota(jnp.int32, (BM, VEC), 1) == 0).astype(jnp.float32)
    out_ref[...] = (
        lax.dot_general(wout_ref[...], h, (((1,), (1,)), ((), ())),
                        preferred_element_type=jnp.float32)
        + lax.dot_general(bout_ref[...], e0, (((1,), (1,)), ((), ())),
                          preferred_element_type=jnp.float32))


def kernel(x, embeddings, W_in, b_in, W_out, b_out):
    counts = _sc_counts(x)
    t = pl.pallas_call(
        _prep_body,
        out_shape=jax.ShapeDtypeStruct((VPAD, VEC), jnp.float32),
    )(embeddings, W_in, b_in.reshape(1, VEC))
    bout_col = jnp.pad(b_out.reshape(VOCAB, 1), ((0, 0), (0, VEC - 1)))
    out_t = pl.pallas_call(
        _main_body,
        grid=(B // BM,),
        in_specs=[
            pl.BlockSpec((BM, 8, 128), lambda i: (i, 0, 0)),
            pl.BlockSpec((VPAD, VEC), lambda i: (0, 0)),
            pl.BlockSpec((VOCAB, VEC), lambda i: (0, 0)),
            pl.BlockSpec((VOCAB, VEC), lambda i: (0, 0)),
        ],
        out_specs=pl.BlockSpec((VOCAB, BM), lambda i: (0, i)),
        out_shape=jax.ShapeDtypeStruct((VOCAB, B), jnp.float32),
    )(counts.reshape(B, 8, 128), t, W_out, bout_col)
    return out_t.T


# speedup vs baseline: 97.7757x; 1.3940x over previous
import functools

import jax
import jax.numpy as jnp
from jax import lax
from jax.experimental import pallas as pl
from jax.experimental.pallas import tpu as pltpu
from jax.experimental.pallas import tpu_sc as plsc

VOCAB = 1000
VPAD = 1024
EMB = 128
VEC = 64
B = 16384
L = 200

NC = 2
NS = 16
LN = 16
NW = NC * NS
ROWS_PER_W = B // NW
R = 32
NCHUNK = ROWS_PER_W // R
CV = R * VPAD

_mesh = plsc.VectorSubcoreMesh(core_axis_name="c", subcore_axis_name="s")


@functools.partial(
    pl.kernel,
    out_type=jax.ShapeDtypeStruct((B * VPAD,), jnp.float32),
    mesh=_mesh,
    scratch_types=[
        pltpu.VMEM((R, L), jnp.int32),
        pltpu.VMEM((R, L), jnp.int32),
        pltpu.VMEM((CV,), jnp.float32),
        pltpu.VMEM((CV,), jnp.float32),
        pltpu.SemaphoreType.DMA,
        pltpu.SemaphoreType.DMA,
    ],
    compiler_params=pltpu.CompilerParams(needs_layout_passes=False),
)
def _sc_counts(x_hbm, out_hbm, idx0, idx1, cnt0, cnt1, sem0, sem1):
    wid = lax.axis_index("s") * NC + lax.axis_index("c")
    ones = jnp.ones((LN,), jnp.float32)
    zeros = jnp.zeros((LN,), jnp.float32)
    lane = lax.iota(jnp.int32, LN)
    tail_mask = lane >= 8
    idx_b = (idx0, idx1)
    cnt_b = (cnt0, cnt1)
    sem_b = (sem0, sem1)

    def zbody(i, carry):
        cnt0[pl.ds(i * LN, LN)] = zeros
        cnt1[pl.ds(i * LN, LN)] = zeros
        return carry

    lax.fori_loop(0, CV // LN, zbody, 0)

    bias_idx0 = lane * VPAD + VOCAB
    bias_idx1 = bias_idx0 + LN * VPAD
    for cbuf in (cnt0, cnt1):
        plsc.store_scatter(cbuf, [bias_idx0], ones)
        plsc.store_scatter(cbuf, [bias_idx1], ones)

    RPI = 2

    def rows_pass(slot, r0, value_vec, add):
        xs, masks = [], []
        for rr in range(RPI):
            r = r0 + rr
            roff = r * VPAD
            for j in range(13):
                if j < 12:
                    xs.append(idx_b[slot][r, pl.ds(j * LN, LN)] + roff)
                    masks.append(None)
                else:
                    xs.append(idx_b[slot][r, pl.ds(L - LN, LN)] + roff)
                    masks.append(tail_mask)
        for fidx, mask in zip(xs, masks):
            if add:
                plsc.addupdate_scatter(cnt_b[slot], [fidx], value_vec, mask=mask)
            else:
                plsc.store_scatter(cnt_b[slot], [fidx], zeros, mask=mask)

    def add_pass(slot):
        def body(i, carry):
            rows_pass(slot, i * RPI, ones, True)
            return carry
        lax.fori_loop(0, R // RPI, body, 0)

    def zero_pass(slot):
        def body(i, carry):
            rows_pass(slot, i * RPI, zeros, False)
            return carry
        lax.fori_loop(0, R // RPI, body, 0)

    def out_copy(slot, g):
        row0 = wid * ROWS_PER_W + g * R
        return pltpu.make_async_copy(
            cnt_b[slot], out_hbm.at[pl.ds(row0 * VPAD, CV)], sem_b[slot])

    def process(slot, g):
        row0 = wid * ROWS_PER_W + g * R
        pltpu.sync_copy(x_hbm.at[pl.ds(row0, R)], idx_b[slot])
        add_pass(slot)
        out_copy(slot, g).start()

    process(0, 0)
    process(1, 1)

    def chunk_body(g, carry):
        def for_slot(slot):
            @pl.when(lax.rem(g, 2) == slot)
            def _():
                out_copy(slot, g - 2).wait()
                zero_pass(slot)
                process(slot, g)
        for_slot(0)
        for_slot(1)
        return carry

    lax.fori_loop(2, NCHUNK, chunk_body, 0)
    out_copy(0, NCHUNK - 2).wait()
    out_copy(1, NCHUNK - 1).wait()


def _prep_body(emb_ref, win_ref, bin_ref, t_ref):
    t_ref[pl.ds(0, VOCAB), :] = lax.dot_general(
        emb_ref[...], win_ref[...], (((1,), (1,)), ((), ())),
        preferred_element_type=jnp.float32)
    t_ref[pl.ds(VOCAB, VPAD - VOCAB), :] = jnp.zeros(
        (VPAD - VOCAB, VEC), jnp.float32)
    t_ref[pl.ds(VOCAB, 1), :] = bin_ref[...]


BM = 1024


def _main_body(cnt_ref, t_ref, wout_ref, bout_ref, out_ref):
    h = jnp.dot(cnt_ref[:, 0, :], t_ref[pl.ds(0, 128), :],
                preferred_element_type=jnp.float32)
    for j in range(1, 8):
        h = h + jnp.dot(cnt_ref[:, j, :], t_ref[pl.ds(j * 128, 128), :],
                        preferred_element_type=jnp.float32)
    e0 = (lax.broadcasted_iota(jnp.int32, (BM, VEC), 1) == 0).astype(jnp.float32)
    out_ref[...] = (
        lax.dot_general(wout_ref[...], h, (((1,), (1,)), ((), ())),
                        preferred_element_type=jnp.float32)
        + lax.dot_general(bout_ref[...], e0, (((1,), (1,)), ((), ())),
                          preferred_element_type=jnp.float32))


def kernel(x, embeddings, W_in, b_in, W_out, b_out):
    counts = _sc_counts(x)
    t = pl.pallas_call(
        _prep_body,
        out_shape=jax.ShapeDtypeStruct((VPAD, VEC), jnp.float32),
    )(embeddings, W_in, b_in.reshape(1, VEC))
    bout_col = jnp.pad(b_out.reshape(VOCAB, 1), ((0, 0), (0, VEC - 1)))
    out_t = pl.pallas_call(
        _main_body,
        grid=(B // BM,),
        in_specs=[
            pl.BlockSpec((BM, 8, 128), lambda i: (i, 0, 0)),
            pl.BlockSpec((VPAD, VEC), lambda i: (0, 0)),
            pl.BlockSpec((VOCAB, VEC), lambda i: (0, 0)),
            pl.BlockSpec((VOCAB, VEC), lambda i: (0, 0)),
        ],
        out_specs=pl.BlockSpec((VOCAB, BM), lambda i: (0, i)),
        out_shape=jax.ShapeDtypeStruct((VOCAB, B), jnp.float32),
    )(counts.reshape(B, 8, 128), t, W_out, bout_col)
    return out_t.T


# speedup vs baseline: 98.4066x; 1.0065x over previous
import functools

import jax
import jax.numpy as jnp
from jax import lax
from jax.experimental import pallas as pl
from jax.experimental.pallas import tpu as pltpu
from jax.experimental.pallas import tpu_sc as plsc

VOCAB = 1000
VPAD = 1024
EMB = 128
VEC = 64
B = 16384
L = 200

NC = 2
NS = 16
LN = 16
NW = NC * NS
R = 32
CV = R * VPAD
BH = B // 2

_mesh = plsc.VectorSubcoreMesh(core_axis_name="c", subcore_axis_name="s")


def _make_sc_counts(nrows):
    rows_per_w = nrows // NW
    nchunk = rows_per_w // R

    @functools.partial(
        pl.kernel,
        out_type=jax.ShapeDtypeStruct((nrows * VPAD,), jnp.float32),
        mesh=_mesh,
        scratch_types=[
            pltpu.VMEM((R, L), jnp.int32),
            pltpu.VMEM((R, L), jnp.int32),
            pltpu.VMEM((CV,), jnp.float32),
            pltpu.VMEM((CV,), jnp.float32),
            pltpu.SemaphoreType.DMA,
            pltpu.SemaphoreType.DMA,
        ],
        compiler_params=pltpu.CompilerParams(needs_layout_passes=False),
    )
    def sc_counts(x_hbm, out_hbm, idx0, idx1, cnt0, cnt1, sem0, sem1):
        wid = lax.axis_index("s") * NC + lax.axis_index("c")
        ones = jnp.ones((LN,), jnp.float32)
        zeros = jnp.zeros((LN,), jnp.float32)
        lane = lax.iota(jnp.int32, LN)
        tail_mask = lane >= 8
        idx_b = (idx0, idx1)
        cnt_b = (cnt0, cnt1)
        sem_b = (sem0, sem1)

        def zbody(i, carry):
            cnt0[pl.ds(i * LN, LN)] = zeros
            cnt1[pl.ds(i * LN, LN)] = zeros
            return carry

        lax.fori_loop(0, CV // LN, zbody, 0)

        bias_idx0 = lane * VPAD + VOCAB
        bias_idx1 = bias_idx0 + LN * VPAD
        for cbuf in (cnt0, cnt1):
            plsc.store_scatter(cbuf, [bias_idx0], ones)
            plsc.store_scatter(cbuf, [bias_idx1], ones)

        RPI = 2

        def rows_pass(slot, r0, value_vec, add):
            xs, masks = [], []
            for rr in range(RPI):
                r = r0 + rr
                roff = r * VPAD
                for j in range(13):
                    if j < 12:
                        xs.append(idx_b[slot][r, pl.ds(j * LN, LN)] + roff)
                        masks.append(None)
                    else:
                        xs.append(idx_b[slot][r, pl.ds(L - LN, LN)] + roff)
                        masks.append(tail_mask)
            for fidx, mask in zip(xs, masks):
                if add:
                    plsc.addupdate_scatter(cnt_b[slot], [fidx], value_vec,
                                           mask=mask)
                else:
                    plsc.store_scatter(cnt_b[slot], [fidx], zeros, mask=mask)

        def add_pass(slot):
            def body(i, carry):
                rows_pass(slot, i * RPI, ones, True)
                return carry
            lax.fori_loop(0, R // RPI, body, 0)

        def zero_pass(slot):
            def body(i, carry):
                rows_pass(slot, i * RPI, zeros, False)
                return carry
            lax.fori_loop(0, R // RPI, body, 0)

        def out_copy(slot, g):
            row0 = wid * rows_per_w + g * R
            return pltpu.make_async_copy(
                cnt_b[slot], out_hbm.at[pl.ds(row0 * VPAD, CV)], sem_b[slot])

        def process(slot, g):
            row0 = wid * rows_per_w + g * R
            pltpu.sync_copy(x_hbm.at[pl.ds(row0, R)], idx_b[slot])
            add_pass(slot)
            out_copy(slot, g).start()

        process(0, 0)
        process(1, 1)

        def chunk_body(g, carry):
            def for_slot(slot):
                @pl.when(lax.rem(g, 2) == slot)
                def _():
                    out_copy(slot, g - 2).wait()
                    zero_pass(slot)
                    process(slot, g)
            for_slot(0)
            for_slot(1)
            return carry

        lax.fori_loop(2, nchunk, chunk_body, 0)
        out_copy(0, nchunk - 2).wait()
        out_copy(1, nchunk - 1).wait()

    return sc_counts


_sc_counts_half = _make_sc_counts(BH)


def _prep_body(emb_ref, win_ref, bin_ref, t_ref):
    t_ref[pl.ds(0, VOCAB), :] = lax.dot_general(
        emb_ref[...], win_ref[...], (((1,), (1,)), ((), ())),
        preferred_element_type=jnp.float32)
    t_ref[pl.ds(VOCAB, VPAD - VOCAB), :] = jnp.zeros(
        (VPAD - VOCAB, VEC), jnp.float32)
    t_ref[pl.ds(VOCAB, 1), :] = bin_ref[...]


BM = 1024
NBH = BH // BM


def _compute_block(cnt_ref, t_ref, wout_ref, bout_ref, out_ref):
    h = jnp.dot(cnt_ref[:, 0, :], t_ref[pl.ds(0, 128), :],
                preferred_element_type=jnp.float32)
    for j in range(1, 8):
        h = h + jnp.dot(cnt_ref[:, j, :], t_ref[pl.ds(j * 128, 128), :],
                        preferred_element_type=jnp.float32)
    e0 = (lax.broadcasted_iota(jnp.int32, (BM, VEC), 1) == 0).astype(jnp.float32)
    out_ref[...] = (
        lax.dot_general(wout_ref[...], h, (((1,), (1,)), ((), ())),
                        preferred_element_type=jnp.float32)
        + lax.dot_general(bout_ref[...], e0, (((1,), (1,)), ((), ())),
                          preferred_element_type=jnp.float32))


def _main_a_body(cnt_ref, t_ref, wout_ref, bout_ref, out_ref):
    _compute_block(cnt_ref, t_ref, wout_ref, bout_ref, out_ref)


def _main_b_body(alias_ref, cnt_ref, t_ref, wout_ref, bout_ref, out_ref):
    del alias_ref
    _compute_block(cnt_ref, t_ref, wout_ref, bout_ref, out_ref)


_weight_specs = [
    pl.BlockSpec((VPAD, VEC), lambda i: (0, 0)),
    pl.BlockSpec((VOCAB, VEC), lambda i: (0, 0)),
    pl.BlockSpec((VOCAB, VEC), lambda i: (0, 0)),
]


def kernel(x, embeddings, W_in, b_in, W_out, b_out):
    counts_a = _sc_counts_half(x[:BH])
    counts_b = _sc_counts_half(x[BH:])
    t = pl.pallas_call(
        _prep_body,
        out_shape=jax.ShapeDtypeStruct((VPAD, VEC), jnp.float32),
    )(embeddings, W_in, b_in.reshape(1, VEC))
    bout_col = jnp.pad(b_out.reshape(VOCAB, 1), ((0, 0), (0, VEC - 1)))
    out_shape = jax.ShapeDtypeStruct((VOCAB, B), jnp.float32)
    out_a = pl.pallas_call(
        _main_a_body,
        grid=(NBH,),
        in_specs=[pl.BlockSpec((BM, 8, 128), lambda i: (i, 0, 0))]
        + _weight_specs,
        out_specs=pl.BlockSpec((VOCAB, BM), lambda i: (0, i)),
        out_shape=out_shape,
    )(counts_a.reshape(BH, 8, 128), t, W_out, bout_col)
    out_t = pl.pallas_call(
        _main_b_body,
        grid=(NBH,),
        in_specs=[pl.BlockSpec(memory_space=pltpu.MemorySpace.HBM)]
        + [pl.BlockSpec((BM, 8, 128), lambda i: (i, 0, 0))]
        + _weight_specs,
        out_specs=pl.BlockSpec((VOCAB, BM), lambda i: (0, i + NBH)),
        out_shape=out_shape,
        input_output_aliases={0: 0},
    )(out_a, counts_b.reshape(BH, 8, 128), t, W_out, bout_col)
    return out_t.T


# speedup vs baseline: 111.1564x; 1.1296x over previous
import functools

import jax
import jax.numpy as jnp
from jax import lax
from jax.experimental import pallas as pl
from jax.experimental.pallas import tpu as pltpu
from jax.experimental.pallas import tpu_sc as plsc

VOCAB = 1000
VPAD = 1024
EMB = 128
VEC = 64
B = 16384
L = 200

NC = 2
NS = 16
LN = 16
NW = NC * NS
R = 32
CV = R * VPAD
BH = B // 2

_mesh = plsc.VectorSubcoreMesh(core_axis_name="c", subcore_axis_name="s")


def _make_sc_counts(nrows, row_off):
    rows_per_w = nrows // NW
    nchunk = rows_per_w // R

    @functools.partial(
        pl.kernel,
        out_type=jax.ShapeDtypeStruct((nrows * VPAD,), jnp.float32),
        mesh=_mesh,
        scratch_types=[
            pltpu.VMEM((R, L), jnp.int32),
            pltpu.VMEM((R, L), jnp.int32),
            pltpu.VMEM((CV,), jnp.float32),
            pltpu.VMEM((CV,), jnp.float32),
            pltpu.SemaphoreType.DMA,
            pltpu.SemaphoreType.DMA,
        ],
        compiler_params=pltpu.CompilerParams(needs_layout_passes=False),
    )
    def sc_counts(x_hbm, out_hbm, idx0, idx1, cnt0, cnt1, sem0, sem1):
        wid = lax.axis_index("s") * NC + lax.axis_index("c")
        ones = jnp.ones((LN,), jnp.float32)
        zeros = jnp.zeros((LN,), jnp.float32)
        lane = lax.iota(jnp.int32, LN)
        tail_mask = lane >= 8
        idx_b = (idx0, idx1)
        cnt_b = (cnt0, cnt1)
        sem_b = (sem0, sem1)

        ZU = 8

        def zbody(i, carry):
            for u in range(ZU):
                cnt0[pl.ds((i * ZU + u) * LN, LN)] = zeros
                cnt1[pl.ds((i * ZU + u) * LN, LN)] = zeros
            return carry

        lax.fori_loop(0, CV // (LN * ZU), zbody, 0)

        bias_idx0 = lane * VPAD + VOCAB
        bias_idx1 = bias_idx0 + LN * VPAD
        for cbuf in (cnt0, cnt1):
            plsc.store_scatter(cbuf, [bias_idx0], ones)
            plsc.store_scatter(cbuf, [bias_idx1], ones)

        RPI = 2

        def rows_pass(slot, r0, value_vec, add):
            xs, masks = [], []
            for rr in range(RPI):
                r = r0 + rr
                roff = r * VPAD
                for j in range(13):
                    if j < 12:
                        xs.append(idx_b[slot][r, pl.ds(j * LN, LN)] + roff)
                        masks.append(None)
                    else:
                        xs.append(idx_b[slot][r, pl.ds(L - LN, LN)] + roff)
                        masks.append(tail_mask)
            for fidx, mask in zip(xs, masks):
                if add:
                    plsc.addupdate_scatter(cnt_b[slot], [fidx], value_vec,
                                           mask=mask)
                else:
                    plsc.store_scatter(cnt_b[slot], [fidx], zeros, mask=mask)

        def add_pass(slot):
            def body(i, carry):
                rows_pass(slot, i * RPI, ones, True)
                return carry
            lax.fori_loop(0, R // RPI, body, 0)

        def zero_pass(slot):
            def body(i, carry):
                rows_pass(slot, i * RPI, zeros, False)
                return carry
            lax.fori_loop(0, R // RPI, body, 0)

        def out_copy(slot, g):
            row0 = wid * rows_per_w + g * R
            return pltpu.make_async_copy(
                cnt_b[slot], out_hbm.at[pl.ds(row0 * VPAD, CV)], sem_b[slot])

        def process(slot, g):
            row0 = wid * rows_per_w + g * R
            pltpu.sync_copy(x_hbm.at[pl.ds(row_off + row0, R)], idx_b[slot])
            add_pass(slot)
            out_copy(slot, g).start()

        process(0, 0)
        process(1, 1)

        def chunk_body(g, carry):
            def for_slot(slot):
                @pl.when(lax.rem(g, 2) == slot)
                def _():
                    out_copy(slot, g - 2).wait()
                    zero_pass(slot)
                    process(slot, g)
            for_slot(0)
            for_slot(1)
            return carry

        lax.fori_loop(2, nchunk, chunk_body, 0)
        out_copy(0, nchunk - 2).wait()
        out_copy(1, nchunk - 1).wait()

    return sc_counts


_sc_counts_a = _make_sc_counts(BH, 0)
_sc_counts_b = _make_sc_counts(BH, BH)


def _prep_body(emb_ref, win_ref, bin_ref, t_ref):
    t_ref[pl.ds(0, VOCAB), :] = lax.dot_general(
        emb_ref[...], win_ref[...], (((1,), (1,)), ((), ())),
        preferred_element_type=jnp.float32)
    t_ref[pl.ds(VOCAB, VPAD - VOCAB), :] = jnp.zeros(
        (VPAD - VOCAB, VEC), jnp.float32)
    t_ref[pl.ds(VOCAB, 1), :] = bin_ref[...]


BM = 1024
NBH = BH // BM


def _compute_block(cnt_ref, t_ref, wout_ref, bout_ref, out_ref):
    h = jnp.dot(cnt_ref[:, 0, :], t_ref[pl.ds(0, 128), :],
                preferred_element_type=jnp.float32)
    for j in range(1, 8):
        h = h + jnp.dot(cnt_ref[:, j, :], t_ref[pl.ds(j * 128, 128), :],
                        preferred_element_type=jnp.float32)
    e0 = (lax.broadcasted_iota(jnp.int32, (BM, VEC), 1) == 0).astype(jnp.float32)
    out_ref[...] = (
        lax.dot_general(wout_ref[...], h, (((1,), (1,)), ((), ())),
                        preferred_element_type=jnp.float32)
        + lax.dot_general(bout_ref[...], e0, (((1,), (1,)), ((), ())),
                          preferred_element_type=jnp.float32))


def _main_a_body(cnt_ref, t_ref, wout_ref, bout_ref, out_ref):
    _compute_block(cnt_ref, t_ref, wout_ref, bout_ref, out_ref)


def _main_b_body(alias_ref, cnt_ref, t_ref, wout_ref, bout_ref, out_ref):
    del alias_ref
    _compute_block(cnt_ref, t_ref, wout_ref, bout_ref, out_ref)


_weight_specs = [
    pl.BlockSpec((VPAD, VEC), lambda i: (0, 0)),
    pl.BlockSpec((VOCAB, VEC), lambda i: (0, 0)),
    pl.BlockSpec((VOCAB, VEC), lambda i: (0, 0)),
]


def kernel(x, embeddings, W_in, b_in, W_out, b_out):
    counts_a = _sc_counts_a(x)
    counts_b = _sc_counts_b(x)
    t = pl.pallas_call(
        _prep_body,
        out_shape=jax.ShapeDtypeStruct((VPAD, VEC), jnp.float32),
    )(embeddings, W_in, b_in.reshape(1, VEC))
    bout_col = jnp.pad(b_out.reshape(VOCAB, 1), ((0, 0), (0, VEC - 1)))
    out_shape = jax.ShapeDtypeStruct((VOCAB, B), jnp.float32)
    out_a = pl.pallas_call(
        _main_a_body,
        grid=(NBH,),
        in_specs=[pl.BlockSpec((BM, 8, 128), lambda i: (i, 0, 0))]
        + _weight_specs,
        out_specs=pl.BlockSpec((VOCAB, BM), lambda i: (0, i)),
        out_shape=out_shape,
    )(counts_a.reshape(BH, 8, 128), t, W_out, bout_col)
    out_t = pl.pallas_call(
        _main_b_body,
        grid=(NBH,),
        in_specs=[pl.BlockSpec(memory_space=pltpu.MemorySpace.HBM)]
        + [pl.BlockSpec((BM, 8, 128), lambda i: (i, 0, 0))]
        + _weight_specs,
        out_specs=pl.BlockSpec((VOCAB, BM), lambda i: (0, i + NBH)),
        out_shape=out_shape,
        input_output_aliases={0: 0},
    )(out_a, counts_b.reshape(BH, 8, 128), t, W_out, bout_col)
    return out_t.T
